# baseline (speedup 1.0000x reference)
import functools

import jax
import jax.numpy as jnp
from jax import lax
from jax.experimental import pallas as pl
from jax.experimental.pallas import tpu as pltpu
from jax.experimental.pallas import tpu_sc as plsc

_SC_PARAMS = pltpu.CompilerParams(needs_layout_passes=False)

N_SUBCORES = 16
N_CORES = 2
LANES = 16


def _sc_mesh():
    return plsc.VectorSubcoreMesh(
        core_axis_name="c", subcore_axis_name="s",
        num_cores=N_CORES, num_subcores=N_SUBCORES)


def _make_deg_c_kernel(E, NPR):
    EPT = E // N_SUBCORES
    assert EPT * N_SUBCORES == E and EPT % LANES == 0
    NITER = EPT // LANES

    @functools.partial(
        pl.kernel,
        out_type=(
            jax.ShapeDtypeStruct((NPR, 128), jnp.float32),
            jax.ShapeDtypeStruct((NPR, 128), jnp.float32),
        ),
        mesh=_sc_mesh(),
        compiler_params=_SC_PARAMS,
        scratch_types=[
            pltpu.VMEM((EPT,), jnp.int32),
            pltpu.VMEM((EPT,), jnp.int32),
            pltpu.VMEM((NPR, 128), jnp.float32),
            pltpu.VMEM((NPR, 128), jnp.float32),
            pltpu.VMEM((NPR,), jnp.int32),
            pltpu.VMEM_SHARED((NPR, 128), jnp.float32),
            pltpu.VMEM_SHARED((NPR, 128), jnp.float32),
        ],
    )
    def deg_c_kernel(src_hbm, dsts_hbm, deg_hbm, c_hbm,
                     dst_v, src_v, part_v, rec_v, rows_v, sh_deg, sh_c):
        cid = lax.axis_index("c")
        sid = lax.axis_index("s")
        iota = lax.iota(jnp.int32, LANES)
        ones = jnp.ones((LANES,), jnp.float32)
        zeros = jnp.zeros((LANES,), jnp.float32)
        base = sid * EPT

        def _fill_rows(i, _):
            rows_v[pl.ds(i * LANES, LANES)] = iota + i * LANES
            return 0
        lax.fori_loop(0, NPR // LANES, _fill_rows, 0)

        def _zero_part(k, _):
            part_v[k >> 3, pl.ds((k & 7) * LANES, LANES)] = zeros
            return 0
        nvec = NPR * 8
        lax.fori_loop(0, nvec, _zero_part, 0)

        @pl.when(sid == 0)
        def _():
            pltpu.sync_copy(part_v, sh_deg)
            pltpu.sync_copy(part_v, sh_c)

        plsc.subcore_barrier()

        pltpu.sync_copy(dsts_hbm.at[pl.ds(base, EPT)], dst_v)
        pltpu.sync_copy(src_hbm.at[pl.ds(base, EPT)], src_v)

        def _deg_step(e, _):
            dv = dst_v[pl.ds(e * LANES, LANES)]
            plsc.addupdate_scatter(part_v, [dv >> 7, dv & 127], ones)
            return 0
        lax.fori_loop(0, NITER, _deg_step, 0)

        pltpu.sync_copy(part_v, sh_deg.at[rows_v], add=True)
        plsc.subcore_barrier()

        pltpu.sync_copy(sh_deg, rec_v)

        def _recip(k, _):
            v = rec_v[k >> 3, pl.ds((k & 7) * LANES, LANES)]
            rec_v[k >> 3, pl.ds((k & 7) * LANES, LANES)] = 1.0 / jnp.maximum(v, 1.0)
            return 0
        lax.fori_loop(0, nvec, _recip, 0)

        lax.fori_loop(0, nvec, _zero_part, 0)

        def _c_step(e, _):
            dv = dst_v[pl.ds(e * LANES, LANES)]
            sv = src_v[pl.ds(e * LANES, LANES)]
            rv = plsc.load_gather(rec_v, [dv >> 7, dv & 127])
            plsc.addupdate_scatter(part_v, [sv >> 7, sv & 127], rv)
            return 0
        lax.fori_loop(0, NITER, _c_step, 0)

        pltpu.sync_copy(part_v, sh_c.at[rows_v], add=True)
        plsc.subcore_barrier()

        NCHUNK = NPR // 8

        @pl.when((cid == 0) & (sid < NCHUNK))
        def _():
            pltpu.sync_copy(sh_deg.at[pl.ds(sid * 8, 8)],
                            part_v.at[pl.ds(0, 8)])
            pltpu.sync_copy(part_v.at[pl.ds(0, 8)],
                            deg_hbm.at[pl.ds(sid * 8, 8)])
            pltpu.sync_copy(sh_c.at[pl.ds(sid * 8, 8)],
                            part_v.at[pl.ds(0, 8)])
            pltpu.sync_copy(part_v.at[pl.ds(0, 8)],
                            c_hbm.at[pl.ds(sid * 8, 8)])

    return deg_c_kernel


def _make_spmm_kernel(ROWS_PER_W, NPR, H):
    NW = N_CORES * N_SUBCORES
    NP = NPR * 128
    RPT = NP // 128 // N_SUBCORES

    @functools.partial(
        pl.kernel,
        out_type=jax.ShapeDtypeStruct((N_CORES, NP, H), jnp.float32),
        mesh=_sc_mesh(),
        compiler_params=_SC_PARAMS,
        scratch_types=[
            pltpu.VMEM((2, 16, 128), jnp.int32),
            pltpu.VMEM((2, 16, 128), jnp.int32),
            pltpu.VMEM((128, H), jnp.float32),
            pltpu.VMEM((128, H), jnp.float32),
            pltpu.SemaphoreType.DMA,
            pltpu.SemaphoreType.DMA,
            pltpu.SemaphoreType.DMA,
            pltpu.SemaphoreType.DMA,
            pltpu.SemaphoreType.DMA,
            pltpu.VMEM_SHARED((NP, H), jnp.float32),
        ],
    )
    def spmm_kernel(src3_hbm, dst3_hbm, g_hbm, m_hbm,
                    src_v, dst_v, rows_a, rows_b,
                    sem_a, sem_b, sem_i, sem_sa, sem_sb, sh_m):
        cid = lax.axis_index("c")
        sid = lax.axis_index("s")
        w = sid * N_CORES + cid
        zeros = jnp.zeros((LANES,), jnp.float32)

        def _zero_buf(k, _):
            rows_a[k >> 3, pl.ds((k & 7) * LANES, LANES)] = zeros
            return 0
        lax.fori_loop(0, 128 * 8, _zero_buf, 0)
        for k in range(RPT):
            pltpu.sync_copy(rows_a, sh_m.at[pl.ds((sid * RPT + k) * 128, 128)])

        plsc.subcore_barrier()

        r0 = w * ROWS_PER_W
        NCH = ROWS_PER_W // 16
        nhalf = 8

        def _stage(ch, buf):
            cbase = r0 + ch * 16
            pltpu.async_copy(src3_hbm.at[pl.ds(cbase, 16)],
                             src_v.at[buf], sem_i)
            pltpu.async_copy(dst3_hbm.at[pl.ds(cbase, 16)],
                             dst_v.at[buf], sem_i)

        def _drain_stage(buf):
            pltpu.make_async_copy(src3_hbm.at[pl.ds(0, 16)],
                                  src_v.at[buf], sem_i).wait()
            pltpu.make_async_copy(dst3_hbm.at[pl.ds(0, 16)],
                                  dst_v.at[buf], sem_i).wait()

        _stage(0, 0)
        for ch in range(NCH):
            b = ch % 2
            sv = src_v.at[b]
            dv = dst_v.at[b]
            _drain_stage(b)
            if ch + 1 < NCH:
                _stage(ch + 1, 1 - b)
            pltpu.async_copy(g_hbm.at[sv.at[0]], rows_a, sem_a)
            pltpu.async_copy(g_hbm.at[sv.at[1]], rows_b, sem_b)

            def _step(j2, _, sv=sv, dv=dv):
                j = j2 * 2
                pltpu.make_async_copy(
                    g_hbm.at[sv.at[j]], rows_a, sem_a).wait()
                pltpu.async_copy(rows_a, sh_m.at[dv.at[j]], sem_sa, add=True)
                pltpu.make_async_copy(
                    g_hbm.at[sv.at[j + 1]], rows_b, sem_b).wait()
                pltpu.async_copy(rows_b, sh_m.at[dv.at[j + 1]], sem_sb,
                                 add=True)

                @pl.when(j2 + 1 < nhalf)
                def _():
                    pltpu.make_async_copy(
                        rows_a, sh_m.at[dv.at[j]], sem_sa).wait()
                    pltpu.async_copy(g_hbm.at[sv.at[j + 2]], rows_a, sem_a)
                    pltpu.make_async_copy(
                        rows_b, sh_m.at[dv.at[j + 1]], sem_sb).wait()
                    pltpu.async_copy(g_hbm.at[sv.at[j + 3]], rows_b, sem_b)
                return 0

            lax.fori_loop(0, nhalf, _step, 0)
            pltpu.make_async_copy(rows_a, sh_m.at[dv.at[14]], sem_sa).wait()
            pltpu.make_async_copy(rows_b, sh_m.at[dv.at[15]], sem_sb).wait()

        plsc.subcore_barrier()

        for k in range(RPT):
            rbase = (sid * RPT + k) * 128
            pltpu.sync_copy(sh_m.at[pl.ds(rbase, 128)], rows_a)
            pltpu.sync_copy(rows_a, m_hbm.at[cid, pl.ds(rbase, 128)])

    return spmm_kernel


def _tc_mm_body(x_ref, wn_ref, ws_ref, g_ref, s_ref):
    xb = x_ref[...]
    g_ref[...] = jnp.dot(xb, wn_ref[...], preferred_element_type=jnp.float32)
    s_ref[...] = jnp.dot(xb, ws_ref[...], preferred_element_type=jnp.float32)


def _make_epilogue(nblocks, blk, N, H, OUT):
    SUB = blk // 8

    def body(s_ref, m0_ref, m1_ref, deg_ref, c_ref, b1_ref,
             w2s_ref, w2n_ref, b2_ref, wh_ref, bh_ref, out_ref,
             u_acc, v_acc):
        i = pl.program_id(0)

        @pl.when(i == 0)
        def _():
            u_acc[...] = jnp.zeros_like(u_acc)
            v_acc[...] = jnp.zeros_like(v_acc)

        msum = m0_ref[0] + m1_ref[0]
        degc = jnp.maximum(deg_ref[...], 1.0)
        h1 = jnp.maximum(s_ref[...] + msum / degc + b1_ref[...], 0.0)
        u_acc[0:1, :] += jnp.sum(h1, axis=0, keepdims=True)
        v_acc[0:1, :] += jnp.sum(h1 * c_ref[...], axis=0, keepdims=True)

        @pl.when(i == nblocks - 1)
        def _():
            u = u_acc[0:1, :]
            v = v_acc[0:1, :]
            pooled = (jnp.dot(u, w2s_ref[...], preferred_element_type=jnp.float32)
                      + jnp.dot(v, w2n_ref[...], preferred_element_type=jnp.float32)
                      ) * (1.0 / N) + b2_ref[...]
            out_ref[...] = (jnp.dot(pooled, wh_ref[...],
                                    preferred_element_type=jnp.float32)
                            + bh_ref[...])

    return body


def kernel(x, edge_index, batch, W1_self, W1_neigh, b1,
           W2_self, W2_neigh, b2, W_head, b_head):
    N, D = x.shape
    E = edge_index.shape[1]
    H = W1_self.shape[0]
    OUT = W_head.shape[0]

    NPR = (N + 127) // 128 + 1
    NP = NPR * 128

    NW = N_CORES * N_SUBCORES
    nrows = (E + 127) // 128
    rows_per_w = (nrows + NW - 1) // NW
    rows_per_w = (rows_per_w + 15) // 16 * 16
    E_pad = rows_per_w * NW * 128
    pad = E_pad - E
    src_flat = edge_index[0]
    dst_flat = edge_index[1]
    pad_ids = lax.iota(jnp.int32, pad)
    junk0 = NP - 128
    src3 = jnp.concatenate(
        [src_flat, pad_ids % N]).reshape(rows_per_w * NW, 128)
    dst3 = jnp.concatenate(
        [dst_flat, junk0 + (pad_ids % 128)]).reshape(rows_per_w * NW, 128)

    deg2d, c2d = _make_deg_c_kernel(E, NPR)(src_flat, dst_flat)

    BLK = 2000
    nblocks = N // BLK
    g, s = pl.pallas_call(
        _tc_mm_body,
        grid=(nblocks,),
        in_specs=[
            pl.BlockSpec((BLK, D), lambda i: (i, 0)),
            pl.BlockSpec((D, H), lambda i: (0, 0)),
            pl.BlockSpec((D, H), lambda i: (0, 0)),
        ],
        out_specs=[
            pl.BlockSpec((BLK, H), lambda i: (i, 0)),
            pl.BlockSpec((BLK, H), lambda i: (i, 0)),
        ],
        out_shape=[
            jax.ShapeDtypeStruct((N, H), jnp.float32),
            jax.ShapeDtypeStruct((N, H), jnp.float32),
        ],
    )(x, W1_neigh.T, W1_self.T)

    m_parts = _make_spmm_kernel(rows_per_w, NPR, H)(src3, dst3, g)

    deg_col = deg2d.reshape(NP)[:N].reshape(N, 1)
    c_col = c2d.reshape(NP)[:N].reshape(N, 1)

    out = pl.pallas_call(
        _make_epilogue(nblocks, BLK, N, H, OUT),
        grid=(nblocks,),
        in_specs=[
            pl.BlockSpec((BLK, H), lambda i: (i, 0)),
            pl.BlockSpec((1, BLK, H), lambda i: (0, i, 0)),
            pl.BlockSpec((1, BLK, H), lambda i: (1, i, 0)),
            pl.BlockSpec((BLK, 1), lambda i: (i, 0)),
            pl.BlockSpec((BLK, 1), lambda i: (i, 0)),
            pl.BlockSpec((1, H), lambda i: (0, 0)),
            pl.BlockSpec((H, H), lambda i: (0, 0)),
            pl.BlockSpec((H, H), lambda i: (0, 0)),
            pl.BlockSpec((1, H), lambda i: (0, 0)),
            pl.BlockSpec((H, OUT), lambda i: (0, 0)),
            pl.BlockSpec((1, OUT), lambda i: (0, 0)),
        ],
        out_specs=pl.BlockSpec((1, OUT), lambda i: (0, 0)),
        out_shape=jax.ShapeDtypeStruct((1, OUT), jnp.float32),
        scratch_shapes=[
            pltpu.VMEM((8, H), jnp.float32),
            pltpu.VMEM((8, H), jnp.float32),
        ],
    )(s, m_parts, m_parts, deg_col, c_col,
      b1.reshape(1, H), W2_self.T, W2_neigh.T, b2.reshape(1, H),
      W_head.T, b_head.reshape(1, OUT))

    return out


# speedup vs baseline: 1.1668x; 1.1668x over previous
import functools

import jax
import jax.numpy as jnp
from jax import lax
from jax.experimental import pallas as pl
from jax.experimental.pallas import tpu as pltpu
from jax.experimental.pallas import tpu_sc as plsc

_SC_PARAMS = pltpu.CompilerParams(needs_layout_passes=False)

N_SUBCORES = 16
N_CORES = 2
LANES = 16


def _sc_mesh():
    return plsc.VectorSubcoreMesh(
        core_axis_name="c", subcore_axis_name="s",
        num_cores=N_CORES, num_subcores=N_SUBCORES)


def _make_deg_c_kernel(E, NPR):
    EPT = E // N_SUBCORES
    assert EPT * N_SUBCORES == E and EPT % LANES == 0
    NITER = EPT // LANES

    E2 = E // N_CORES
    EPT2 = E2 // N_SUBCORES
    assert EPT2 % (5 * LANES) == 0

    @functools.partial(
        pl.kernel,
        out_type=(
            jax.ShapeDtypeStruct((NPR, 128), jnp.float32),
            jax.ShapeDtypeStruct((N_CORES, NPR, 128), jnp.float32),
        ),
        mesh=_sc_mesh(),
        compiler_params=_SC_PARAMS,
        scratch_types=[
            pltpu.VMEM((EPT,), jnp.int32),
            pltpu.VMEM((EPT,), jnp.int32),
            pltpu.VMEM((NPR, 128), jnp.float32),
            pltpu.VMEM((NPR, 128), jnp.float32),
            pltpu.VMEM((NPR,), jnp.int32),
            pltpu.VMEM_SHARED((NPR, 128), jnp.float32),
            pltpu.VMEM_SHARED((NPR, 128), jnp.float32),
        ],
    )
    def deg_c_kernel(src_hbm, dsts_hbm, deg_hbm, c_hbm,
                     dst_v, src_v, part_v, rec_v, rows_v, sh_deg, sh_c):
        cid = lax.axis_index("c")
        sid = lax.axis_index("s")
        iota = lax.iota(jnp.int32, LANES)
        ones = jnp.ones((LANES,), jnp.float32)
        zeros = jnp.zeros((LANES,), jnp.float32)
        base = sid * EPT

        def _fill_rows(i, _):
            rows_v[pl.ds(i * LANES, LANES)] = iota + i * LANES
            return 0
        lax.fori_loop(0, NPR // LANES, _fill_rows, 0)

        def _zero_part(k, _):
            part_v[k >> 3, pl.ds((k & 7) * LANES, LANES)] = zeros
            return 0
        nvec = NPR * 8
        lax.fori_loop(0, nvec, _zero_part, 0)

        @pl.when(sid == 0)
        def _():
            pltpu.sync_copy(part_v, sh_deg)
            pltpu.sync_copy(part_v, sh_c)

        plsc.subcore_barrier()

        pltpu.sync_copy(dsts_hbm.at[pl.ds(base, EPT)], dst_v)
        pltpu.sync_copy(src_hbm.at[pl.ds(base, EPT)], src_v)

        def _deg_step(e2, _):
            for q in range(2):
                dv = dst_v[pl.ds((e2 * 2 + q) * LANES, LANES)]
                plsc.addupdate_scatter(part_v, [dv >> 7, dv & 127], ones)
            return 0
        lax.fori_loop(0, NITER // 2, _deg_step, 0)

        pltpu.sync_copy(part_v, sh_deg.at[rows_v], add=True)
        plsc.subcore_barrier()

        pltpu.sync_copy(sh_deg, rec_v)

        def _recip(k, _):
            v = rec_v[k >> 3, pl.ds((k & 7) * LANES, LANES)]
            rec_v[k >> 3, pl.ds((k & 7) * LANES, LANES)] = 1.0 / jnp.maximum(v, 1.0)
            return 0
        lax.fori_loop(0, nvec, _recip, 0)

        lax.fori_loop(0, nvec, _zero_part, 0)
        base2 = cid * E2 + sid * EPT2
        pltpu.sync_copy(dsts_hbm.at[pl.ds(base2, EPT2)],
                        dst_v.at[pl.ds(0, EPT2)])
        pltpu.sync_copy(src_hbm.at[pl.ds(base2, EPT2)],
                        src_v.at[pl.ds(0, EPT2)])

        def _c_step(e5, _):
            for q in range(5):
                e = e5 * 5 + q
                dv = dst_v[pl.ds(e * LANES, LANES)]
                sv = src_v[pl.ds(e * LANES, LANES)]
                rv = plsc.load_gather(rec_v, [dv >> 7, dv & 127])
                plsc.addupdate_scatter(part_v, [sv >> 7, sv & 127], rv)
            return 0
        lax.fori_loop(0, EPT2 // LANES // 5, _c_step, 0)

        pltpu.sync_copy(part_v, sh_c.at[rows_v], add=True)
        plsc.subcore_barrier()

        NCHUNK = NPR // 8

        @pl.when(sid < NCHUNK)
        def _():
            pltpu.sync_copy(sh_c.at[pl.ds(sid * 8, 8)],
                            part_v.at[pl.ds(0, 8)])
            pltpu.sync_copy(part_v.at[pl.ds(0, 8)],
                            c_hbm.at[cid, pl.ds(sid * 8, 8)])

            @pl.when(cid == 0)
            def _():
                pltpu.sync_copy(sh_deg.at[pl.ds(sid * 8, 8)],
                                rec_v.at[pl.ds(0, 8)])
                pltpu.sync_copy(rec_v.at[pl.ds(0, 8)],
                                deg_hbm.at[pl.ds(sid * 8, 8)])

    return deg_c_kernel


def _make_spmm_kernel(ROWS_PER_W, NPR, H):
    NW = N_CORES * N_SUBCORES
    NP = NPR * 128
    RPT = NP // 128 // N_SUBCORES

    @functools.partial(
        pl.kernel,
        out_type=jax.ShapeDtypeStruct((N_CORES, NP, H), jnp.float32),
        mesh=_sc_mesh(),
        compiler_params=_SC_PARAMS,
        scratch_types=[
            pltpu.VMEM((2, 16, 128), jnp.int32),
            pltpu.VMEM((2, 16, 128), jnp.int32),
            pltpu.VMEM((128, H), jnp.float32),
            pltpu.VMEM((128, H), jnp.float32),
            pltpu.SemaphoreType.DMA,
            pltpu.SemaphoreType.DMA,
            pltpu.SemaphoreType.DMA,
            pltpu.SemaphoreType.DMA,
            pltpu.SemaphoreType.DMA,
            pltpu.VMEM_SHARED((NP, H), jnp.float32),
        ],
    )
    def spmm_kernel(src3_hbm, dst3_hbm, g_hbm, m_hbm,
                    src_v, dst_v, rows_a, rows_b,
                    sem_a, sem_b, sem_i, sem_sa, sem_sb, sh_m):
        cid = lax.axis_index("c")
        sid = lax.axis_index("s")
        w = sid * N_CORES + cid
        zeros = jnp.zeros((LANES,), jnp.float32)

        def _zero_buf(k, _):
            rows_a[k >> 3, pl.ds((k & 7) * LANES, LANES)] = zeros
            return 0
        lax.fori_loop(0, 128 * 8, _zero_buf, 0)
        for k in range(RPT):
            pltpu.sync_copy(rows_a, sh_m.at[pl.ds((sid * RPT + k) * 128, 128)])

        plsc.subcore_barrier()

        r0 = w * ROWS_PER_W
        NCH = ROWS_PER_W // 16
        nhalf = 8

        def _stage(ch, buf):
            cbase = r0 + ch * 16
            pltpu.async_copy(src3_hbm.at[pl.ds(cbase, 16)],
                             src_v.at[buf], sem_i)
            pltpu.async_copy(dst3_hbm.at[pl.ds(cbase, 16)],
                             dst_v.at[buf], sem_i)

        def _drain_stage(buf):
            pltpu.make_async_copy(src3_hbm.at[pl.ds(0, 16)],
                                  src_v.at[buf], sem_i).wait()
            pltpu.make_async_copy(dst3_hbm.at[pl.ds(0, 16)],
                                  dst_v.at[buf], sem_i).wait()

        _stage(0, 0)
        for ch in range(NCH):
            b = ch % 2
            sv = src_v.at[b]
            dv = dst_v.at[b]
            _drain_stage(b)
            if ch + 1 < NCH:
                _stage(ch + 1, 1 - b)
            pltpu.async_copy(g_hbm.at[sv.at[0]], rows_a, sem_a)

            def _step(j2, _, sv=sv, dv=dv):
                j = j2 * 2
                pltpu.make_async_copy(
                    g_hbm.at[sv.at[j]], rows_a, sem_a).wait()
                cp_b = pltpu.async_copy(
                    g_hbm.at[sv.at[j + 1]], rows_b, sem_b)
                pltpu.sync_copy(rows_a, sh_m.at[dv.at[j]], add=True)

                @pl.when(j2 + 1 < nhalf)
                def _():
                    pltpu.async_copy(
                        g_hbm.at[sv.at[j + 2]], rows_a, sem_a)
                cp_b.wait()
                pltpu.sync_copy(rows_b, sh_m.at[dv.at[j + 1]], add=True)
                return 0

            lax.fori_loop(0, nhalf, _step, 0)

        plsc.subcore_barrier()

        for k in range(RPT):
            rbase = (sid * RPT + k) * 128
            pltpu.sync_copy(sh_m.at[pl.ds(rbase, 128)], rows_a)
            pltpu.sync_copy(rows_a, m_hbm.at[cid, pl.ds(rbase, 128)])

    return spmm_kernel


def _tc_mm_body(x_ref, wn_ref, ws_ref, g_ref, s_ref):
    xb = x_ref[...]
    g_ref[...] = jnp.dot(xb, wn_ref[...], preferred_element_type=jnp.float32)
    s_ref[...] = jnp.dot(xb, ws_ref[...], preferred_element_type=jnp.float32)


def _make_epilogue(nblocks, blk, N, H, OUT):
    SUB = blk // 8

    def body(s_ref, m0_ref, m1_ref, deg_ref, c0_ref, c1_ref, b1_ref,
             w2s_ref, w2n_ref, b2_ref, wh_ref, bh_ref, out_ref,
             u_acc, v_acc):
        i = pl.program_id(0)

        @pl.when(i == 0)
        def _():
            u_acc[...] = jnp.zeros_like(u_acc)
            v_acc[...] = jnp.zeros_like(v_acc)

        msum = m0_ref[0] + m1_ref[0]
        degc = jnp.maximum(deg_ref[...], 1.0)
        h1 = jnp.maximum(s_ref[...] + msum / degc + b1_ref[...], 0.0)
        ccol = c0_ref[0] + c1_ref[0]
        u_acc[0:1, :] += jnp.sum(h1, axis=0, keepdims=True)
        v_acc[0:1, :] += jnp.sum(h1 * ccol, axis=0, keepdims=True)

        @pl.when(i == nblocks - 1)
        def _():
            u = u_acc[0:1, :]
            v = v_acc[0:1, :]
            pooled = (jnp.dot(u, w2s_ref[...], preferred_element_type=jnp.float32)
                      + jnp.dot(v, w2n_ref[...], preferred_element_type=jnp.float32)
                      ) * (1.0 / N) + b2_ref[...]
            out_ref[...] = (jnp.dot(pooled, wh_ref[...],
                                    preferred_element_type=jnp.float32)
                            + bh_ref[...])

    return body


def kernel(x, edge_index, batch, W1_self, W1_neigh, b1,
           W2_self, W2_neigh, b2, W_head, b_head):
    N, D = x.shape
    E = edge_index.shape[1]
    H = W1_self.shape[0]
    OUT = W_head.shape[0]

    NPR = (N + 127) // 128 + 1
    NP = NPR * 128

    NW = N_CORES * N_SUBCORES
    nrows = (E + 127) // 128
    rows_per_w = (nrows + NW - 1) // NW
    rows_per_w = (rows_per_w + 15) // 16 * 16
    E_pad = rows_per_w * NW * 128
    pad = E_pad - E
    src_flat = edge_index[0]
    dst_flat = edge_index[1]
    pad_ids = lax.iota(jnp.int32, pad)
    junk0 = NP - 128
    src3 = jnp.concatenate(
        [src_flat, pad_ids % N]).reshape(rows_per_w * NW, 128)
    dst3 = jnp.concatenate(
        [dst_flat, junk0 + (pad_ids % 128)]).reshape(rows_per_w * NW, 128)

    deg2d, c2d = _make_deg_c_kernel(E, NPR)(src_flat, dst_flat)

    BLK = 2000
    nblocks = N // BLK
    g, s = pl.pallas_call(
        _tc_mm_body,
        grid=(nblocks,),
        in_specs=[
            pl.BlockSpec((BLK, D), lambda i: (i, 0)),
            pl.BlockSpec((D, H), lambda i: (0, 0)),
            pl.BlockSpec((D, H), lambda i: (0, 0)),
        ],
        out_specs=[
            pl.BlockSpec((BLK, H), lambda i: (i, 0)),
            pl.BlockSpec((BLK, H), lambda i: (i, 0)),
        ],
        out_shape=[
            jax.ShapeDtypeStruct((N, H), jnp.float32),
            jax.ShapeDtypeStruct((N, H), jnp.float32),
        ],
    )(x, W1_neigh.T, W1_self.T)

    m_parts = _make_spmm_kernel(rows_per_w, NPR, H)(src3, dst3, g)

    deg_col = deg2d.reshape(NP)[:N].reshape(N, 1)
    c_cols = c2d.reshape(N_CORES, NP)[:, :N].reshape(N_CORES, N, 1)

    out = pl.pallas_call(
        _make_epilogue(nblocks, BLK, N, H, OUT),
        grid=(nblocks,),
        in_specs=[
            pl.BlockSpec((BLK, H), lambda i: (i, 0)),
            pl.BlockSpec((1, BLK, H), lambda i: (0, i, 0)),
            pl.BlockSpec((1, BLK, H), lambda i: (1, i, 0)),
            pl.BlockSpec((BLK, 1), lambda i: (i, 0)),
            pl.BlockSpec((1, BLK, 1), lambda i: (0, i, 0)),
            pl.BlockSpec((1, BLK, 1), lambda i: (1, i, 0)),
            pl.BlockSpec((1, H), lambda i: (0, 0)),
            pl.BlockSpec((H, H), lambda i: (0, 0)),
            pl.BlockSpec((H, H), lambda i: (0, 0)),
            pl.BlockSpec((1, H), lambda i: (0, 0)),
            pl.BlockSpec((H, OUT), lambda i: (0, 0)),
            pl.BlockSpec((1, OUT), lambda i: (0, 0)),
        ],
        out_specs=pl.BlockSpec((1, OUT), lambda i: (0, 0)),
        out_shape=jax.ShapeDtypeStruct((1, OUT), jnp.float32),
        scratch_shapes=[
            pltpu.VMEM((8, H), jnp.float32),
            pltpu.VMEM((8, H), jnp.float32),
        ],
    )(s, m_parts, m_parts, deg_col, c_cols, c_cols,
      b1.reshape(1, H), W2_self.T, W2_neigh.T, b2.reshape(1, H),
      W_head.T, b_head.reshape(1, OUT))

    return out


# speedup vs baseline: 1.1716x; 1.0041x over previous
import functools

import jax
import jax.numpy as jnp
from jax import lax
from jax.experimental import pallas as pl
from jax.experimental.pallas import tpu as pltpu
from jax.experimental.pallas import tpu_sc as plsc

_SC_PARAMS = pltpu.CompilerParams(needs_layout_passes=False)

N_SUBCORES = 16
N_CORES = 2
LANES = 16


def _sc_mesh():
    return plsc.VectorSubcoreMesh(
        core_axis_name="c", subcore_axis_name="s",
        num_cores=N_CORES, num_subcores=N_SUBCORES)


def _make_deg_c_kernel(E, NPR):
    EPT = E // N_SUBCORES
    assert EPT * N_SUBCORES == E and EPT % LANES == 0
    NITER = EPT // LANES

    E2 = E // N_CORES
    EPT2 = E2 // N_SUBCORES
    assert EPT2 % (5 * LANES) == 0

    @functools.partial(
        pl.kernel,
        out_type=(
            jax.ShapeDtypeStruct((NPR, 128), jnp.float32),
            jax.ShapeDtypeStruct((N_CORES, NPR, 128), jnp.float32),
        ),
        mesh=_sc_mesh(),
        compiler_params=_SC_PARAMS,
        scratch_types=[
            pltpu.VMEM((EPT,), jnp.int32),
            pltpu.VMEM((EPT,), jnp.int32),
            pltpu.VMEM((NPR, 128), jnp.float32),
            pltpu.VMEM((NPR, 128), jnp.float32),
            pltpu.VMEM((NPR,), jnp.int32),
            pltpu.VMEM_SHARED((NPR, 128), jnp.float32),
            pltpu.VMEM_SHARED((NPR, 128), jnp.float32),
        ],
    )
    def deg_c_kernel(src_hbm, dsts_hbm, r_hbm, c_hbm,
                     dst_v, src_v, part_v, rec_v, rows_v, sh_deg, sh_c):
        cid = lax.axis_index("c")
        sid = lax.axis_index("s")
        iota = lax.iota(jnp.int32, LANES)
        ones = jnp.ones((LANES,), jnp.float32)
        zeros = jnp.zeros((LANES,), jnp.float32)
        base = sid * EPT

        def _fill_rows(i, _):
            rows_v[pl.ds(i * LANES, LANES)] = iota + i * LANES
            return 0
        lax.fori_loop(0, NPR // LANES, _fill_rows, 0)

        def _zero_part(k, _):
            part_v[k >> 3, pl.ds((k & 7) * LANES, LANES)] = zeros
            return 0
        nvec = NPR * 8
        lax.fori_loop(0, nvec, _zero_part, 0)

        @pl.when(sid == 0)
        def _():
            pltpu.sync_copy(part_v, sh_deg)
            pltpu.sync_copy(part_v, sh_c)

        plsc.subcore_barrier()

        pltpu.sync_copy(dsts_hbm.at[pl.ds(base, EPT)], dst_v)
        pltpu.sync_copy(src_hbm.at[pl.ds(base, EPT)], src_v)

        def _deg_step(e2, _):
            for q in range(2):
                dv = dst_v[pl.ds((e2 * 2 + q) * LANES, LANES)]
                plsc.addupdate_scatter(part_v, [dv >> 7, dv & 127], ones)
            return 0
        lax.fori_loop(0, NITER // 2, _deg_step, 0)

        pltpu.sync_copy(part_v, sh_deg.at[rows_v], add=True)
        plsc.subcore_barrier()

        pltpu.sync_copy(sh_deg, rec_v)

        def _recip(k, _):
            v = rec_v[k >> 3, pl.ds((k & 7) * LANES, LANES)]
            rec_v[k >> 3, pl.ds((k & 7) * LANES, LANES)] = 1.0 / jnp.maximum(v, 1.0)
            return 0
        lax.fori_loop(0, nvec, _recip, 0)

        lax.fori_loop(0, nvec, _zero_part, 0)
        base2 = cid * E2 + sid * EPT2
        pltpu.sync_copy(dsts_hbm.at[pl.ds(base2, EPT2)],
                        dst_v.at[pl.ds(0, EPT2)])
        pltpu.sync_copy(src_hbm.at[pl.ds(base2, EPT2)],
                        src_v.at[pl.ds(0, EPT2)])

        def _c_step(e5, _):
            for q in range(5):
                e = e5 * 5 + q
                dv = dst_v[pl.ds(e * LANES, LANES)]
                sv = src_v[pl.ds(e * LANES, LANES)]
                rv = plsc.load_gather(rec_v, [dv >> 7, dv & 127])
                plsc.addupdate_scatter(part_v, [sv >> 7, sv & 127], rv)
            return 0
        lax.fori_loop(0, EPT2 // LANES // 5, _c_step, 0)

        pltpu.sync_copy(part_v, sh_c.at[rows_v], add=True)
        plsc.subcore_barrier()

        NCHUNK = NPR // 8

        @pl.when(sid < NCHUNK)
        def _():
            pltpu.sync_copy(sh_c.at[pl.ds(sid * 8, 8)],
                            part_v.at[pl.ds(0, 8)])
            pltpu.sync_copy(part_v.at[pl.ds(0, 8)],
                            c_hbm.at[cid, pl.ds(sid * 8, 8)])

            @pl.when(cid == 0)
            def _():
                pltpu.sync_copy(rec_v.at[pl.ds(sid * 8, 8)],
                                r_hbm.at[pl.ds(sid * 8, 8)])

    return deg_c_kernel


def _make_spmm_kernel(ROWS_PER_W, NPR, H):
    NW = N_CORES * N_SUBCORES
    NP = NPR * 128
    RPT = NP // 128 // N_SUBCORES

    @functools.partial(
        pl.kernel,
        out_type=jax.ShapeDtypeStruct((N_CORES, NP, H), jnp.float32),
        mesh=_sc_mesh(),
        compiler_params=_SC_PARAMS,
        scratch_types=[
            pltpu.VMEM((2, 16, 128), jnp.int32),
            pltpu.VMEM((2, 16, 128), jnp.int32),
            pltpu.VMEM((128, H), jnp.float32),
            pltpu.VMEM((128, H), jnp.float32),
            pltpu.SemaphoreType.DMA,
            pltpu.SemaphoreType.DMA,
            pltpu.SemaphoreType.DMA,
            pltpu.SemaphoreType.DMA,
            pltpu.SemaphoreType.DMA,
            pltpu.VMEM_SHARED((NP, H), jnp.float32),
        ],
    )
    def spmm_kernel(src3_hbm, dst3_hbm, x_hbm, m_hbm,
                    src_v, dst_v, rows_a, rows_b,
                    sem_a, sem_b, sem_i, sem_sa, sem_sb, sh_m):
        cid = lax.axis_index("c")
        sid = lax.axis_index("s")
        w = sid * N_CORES + cid
        zeros = jnp.zeros((LANES,), jnp.float32)

        def _zero_buf(k, _):
            rows_a[k >> 3, pl.ds((k & 7) * LANES, LANES)] = zeros
            return 0
        lax.fori_loop(0, 128 * 8, _zero_buf, 0)
        for k in range(RPT):
            pltpu.sync_copy(rows_a, sh_m.at[pl.ds((sid * RPT + k) * 128, 128)])

        plsc.subcore_barrier()

        r0 = w * ROWS_PER_W
        NCH = ROWS_PER_W // 16
        nhalf = 8

        def _stage(ch, buf):
            cbase = r0 + ch * 16
            pltpu.async_copy(src3_hbm.at[pl.ds(cbase, 16)],
                             src_v.at[buf], sem_i)
            pltpu.async_copy(dst3_hbm.at[pl.ds(cbase, 16)],
                             dst_v.at[buf], sem_i)

        def _drain_stage(buf):
            pltpu.make_async_copy(src3_hbm.at[pl.ds(0, 16)],
                                  src_v.at[buf], sem_i).wait()
            pltpu.make_async_copy(dst3_hbm.at[pl.ds(0, 16)],
                                  dst_v.at[buf], sem_i).wait()

        _stage(0, 0)
        for ch in range(NCH):
            b = ch % 2
            sv = src_v.at[b]
            dv = dst_v.at[b]
            _drain_stage(b)
            if ch + 1 < NCH:
                _stage(ch + 1, 1 - b)
            pltpu.async_copy(x_hbm.at[sv.at[0]], rows_a, sem_a)

            def _step(j2, _, sv=sv, dv=dv):
                j = j2 * 2
                pltpu.make_async_copy(
                    x_hbm.at[sv.at[j]], rows_a, sem_a).wait()
                cp_b = pltpu.async_copy(
                    x_hbm.at[sv.at[j + 1]], rows_b, sem_b)
                pltpu.sync_copy(rows_a, sh_m.at[dv.at[j]], add=True)

                @pl.when(j2 + 1 < nhalf)
                def _():
                    pltpu.async_copy(
                        x_hbm.at[sv.at[j + 2]], rows_a, sem_a)
                cp_b.wait()
                pltpu.sync_copy(rows_b, sh_m.at[dv.at[j + 1]], add=True)
                return 0

            lax.fori_loop(0, nhalf, _step, 0)

        plsc.subcore_barrier()

        for k in range(RPT):
            rbase = (sid * RPT + k) * 128
            pltpu.sync_copy(sh_m.at[pl.ds(rbase, 128)], rows_a)
            pltpu.sync_copy(rows_a, m_hbm.at[cid, pl.ds(rbase, 128)])

    return spmm_kernel


def _make_epilogue(nblocks, blk, N, H, OUT):
    SUB = blk // 8

    def body(x_ref, m0_ref, m1_ref, r_ref, c0_ref, c1_ref, b1_ref,
             w1s_ref, w1n_ref, w2s_ref, w2n_ref, b2_ref, wh_ref, bh_ref,
             out_ref, u_acc, v_acc):
        i = pl.program_id(0)

        @pl.when(i == 0)
        def _():
            u_acc[...] = jnp.zeros_like(u_acc)
            v_acc[...] = jnp.zeros_like(v_acc)

        mean = (m0_ref[0] + m1_ref[0]) * r_ref[...]
        h1 = jnp.maximum(
            jnp.dot(x_ref[...], w1s_ref[...],
                    preferred_element_type=jnp.float32)
            + jnp.dot(mean, w1n_ref[...],
                      preferred_element_type=jnp.float32)
            + b1_ref[...], 0.0)
        ccol = c0_ref[0] + c1_ref[0]
        u_acc[0:1, :] += jnp.sum(h1, axis=0, keepdims=True)
        v_acc[0:1, :] += jnp.sum(h1 * ccol, axis=0, keepdims=True)

        @pl.when(i == nblocks - 1)
        def _():
            u = u_acc[0:1, :]
            v = v_acc[0:1, :]
            pooled = (jnp.dot(u, w2s_ref[...], preferred_element_type=jnp.float32)
                      + jnp.dot(v, w2n_ref[...], preferred_element_type=jnp.float32)
                      ) * (1.0 / N) + b2_ref[...]
            out_ref[...] = (jnp.dot(pooled, wh_ref[...],
                                    preferred_element_type=jnp.float32)
                            + bh_ref[...])

    return body


def kernel(x, edge_index, batch, W1_self, W1_neigh, b1,
           W2_self, W2_neigh, b2, W_head, b_head):
    N, D = x.shape
    E = edge_index.shape[1]
    H = W1_self.shape[0]
    OUT = W_head.shape[0]

    NPR = (N + 127) // 128 + 1
    NP = NPR * 128

    NW = N_CORES * N_SUBCORES
    nrows = (E + 127) // 128
    rows_per_w = (nrows + NW - 1) // NW
    rows_per_w = (rows_per_w + 15) // 16 * 16
    E_pad = rows_per_w * NW * 128
    pad = E_pad - E
    src_flat = edge_index[0]
    dst_flat = edge_index[1]
    pad_ids = lax.iota(jnp.int32, pad)
    junk0 = NP - 128
    src3 = jnp.concatenate(
        [src_flat, pad_ids % N]).reshape(rows_per_w * NW, 128)
    dst3 = jnp.concatenate(
        [dst_flat, junk0 + (pad_ids % 128)]).reshape(rows_per_w * NW, 128)

    r2d, c2d = _make_deg_c_kernel(E, NPR)(src_flat, dst_flat)

    m_parts = _make_spmm_kernel(rows_per_w, NPR, H)(src3, dst3, x)

    BLK = 2000
    nblocks = N // BLK
    r_col = r2d.reshape(NP)[:N].reshape(N, 1)
    c_cols = c2d.reshape(N_CORES, NP)[:, :N].reshape(N_CORES, N, 1)

    out = pl.pallas_call(
        _make_epilogue(nblocks, BLK, N, H, OUT),
        grid=(nblocks,),
        in_specs=[
            pl.BlockSpec((BLK, D), lambda i: (i, 0)),
            pl.BlockSpec((1, BLK, H), lambda i: (0, i, 0)),
            pl.BlockSpec((1, BLK, H), lambda i: (1, i, 0)),
            pl.BlockSpec((BLK, 1), lambda i: (i, 0)),
            pl.BlockSpec((1, BLK, 1), lambda i: (0, i, 0)),
            pl.BlockSpec((1, BLK, 1), lambda i: (1, i, 0)),
            pl.BlockSpec((1, H), lambda i: (0, 0)),
            pl.BlockSpec((D, H), lambda i: (0, 0)),
            pl.BlockSpec((H, H), lambda i: (0, 0)),
            pl.BlockSpec((H, H), lambda i: (0, 0)),
            pl.BlockSpec((H, H), lambda i: (0, 0)),
            pl.BlockSpec((1, H), lambda i: (0, 0)),
            pl.BlockSpec((H, OUT), lambda i: (0, 0)),
            pl.BlockSpec((1, OUT), lambda i: (0, 0)),
        ],
        out_specs=pl.BlockSpec((1, OUT), lambda i: (0, 0)),
        out_shape=jax.ShapeDtypeStruct((1, OUT), jnp.float32),
        scratch_shapes=[
            pltpu.VMEM((8, H), jnp.float32),
            pltpu.VMEM((8, H), jnp.float32),
        ],
    )(x, m_parts, m_parts, r_col, c_cols, c_cols,
      b1.reshape(1, H), W1_self.T, W1_neigh.T, W2_self.T, W2_neigh.T,
      b2.reshape(1, H), W_head.T, b_head.reshape(1, OUT))

    return out


# speedup vs baseline: 1.2293x; 1.0493x over previous
import functools

import jax
import jax.numpy as jnp
from jax import lax
from jax.experimental import pallas as pl
from jax.experimental.pallas import tpu as pltpu
from jax.experimental.pallas import tpu_sc as plsc

_SC_PARAMS = pltpu.CompilerParams(needs_layout_passes=False)

N_SUBCORES = 16
N_CORES = 2
LANES = 16


def _sc_mesh():
    return plsc.VectorSubcoreMesh(
        core_axis_name="c", subcore_axis_name="s",
        num_cores=N_CORES, num_subcores=N_SUBCORES)


def _make_deg_c_kernel(E, NPR):
    EPT = E // N_SUBCORES
    assert EPT * N_SUBCORES == E and EPT % LANES == 0
    NITER = EPT // LANES

    E2 = E // N_CORES
    EPT2 = E2 // N_SUBCORES
    assert EPT2 % (5 * LANES) == 0

    @functools.partial(
        pl.kernel,
        out_type=(
            jax.ShapeDtypeStruct((NPR, 128), jnp.float32),
            jax.ShapeDtypeStruct((N_CORES, NPR, 128), jnp.float32),
        ),
        mesh=_sc_mesh(),
        compiler_params=_SC_PARAMS,
        scratch_types=[
            pltpu.VMEM((EPT,), jnp.int32),
            pltpu.VMEM((EPT,), jnp.int32),
            pltpu.VMEM((NPR, 128), jnp.float32),
            pltpu.VMEM((NPR, 128), jnp.float32),
            pltpu.VMEM((NPR,), jnp.int32),
            pltpu.VMEM_SHARED((NPR, 128), jnp.float32),
            pltpu.VMEM_SHARED((NPR, 128), jnp.float32),
        ],
    )
    def deg_c_kernel(ef_hbm, r_hbm, c_hbm,
                     dst_v, src_v, part_v, rec_v, rows_v, sh_deg, sh_c):
        cid = lax.axis_index("c")
        sid = lax.axis_index("s")
        iota = lax.iota(jnp.int32, LANES)
        ones = jnp.ones((LANES,), jnp.float32)
        zeros = jnp.zeros((LANES,), jnp.float32)
        base = sid * EPT

        def _fill_rows(i, _):
            rows_v[pl.ds(i * LANES, LANES)] = iota + i * LANES
            return 0
        lax.fori_loop(0, NPR // LANES, _fill_rows, 0)

        def _zero_part(k, _):
            part_v[k >> 3, pl.ds((k & 7) * LANES, LANES)] = zeros
            return 0
        nvec = NPR * 8
        lax.fori_loop(0, nvec, _zero_part, 0)

        @pl.when(sid == 0)
        def _():
            pltpu.sync_copy(part_v, sh_deg)
            pltpu.sync_copy(part_v, sh_c)

        plsc.subcore_barrier()

        pltpu.sync_copy(ef_hbm.at[pl.ds(E + base, EPT)], dst_v)
        pltpu.sync_copy(ef_hbm.at[pl.ds(base, EPT)], src_v)

        def _deg_step(e2, _):
            for q in range(2):
                dv = dst_v[pl.ds((e2 * 2 + q) * LANES, LANES)]
                plsc.addupdate_scatter(part_v, [dv >> 7, dv & 127], ones)
            return 0
        lax.fori_loop(0, NITER // 2, _deg_step, 0)

        pltpu.sync_copy(part_v, sh_deg.at[rows_v], add=True)
        plsc.subcore_barrier()

        pltpu.sync_copy(sh_deg, rec_v)

        def _recip(k, _):
            v = rec_v[k >> 3, pl.ds((k & 7) * LANES, LANES)]
            rec_v[k >> 3, pl.ds((k & 7) * LANES, LANES)] = 1.0 / jnp.maximum(v, 1.0)
            return 0
        lax.fori_loop(0, nvec, _recip, 0)

        lax.fori_loop(0, nvec, _zero_part, 0)
        base2 = cid * E2 + sid * EPT2
        pltpu.sync_copy(ef_hbm.at[pl.ds(E + base2, EPT2)],
                        dst_v.at[pl.ds(0, EPT2)])
        pltpu.sync_copy(ef_hbm.at[pl.ds(base2, EPT2)],
                        src_v.at[pl.ds(0, EPT2)])

        def _c_step(e5, _):
            for q in range(5):
                e = e5 * 5 + q
                dv = dst_v[pl.ds(e * LANES, LANES)]
                sv = src_v[pl.ds(e * LANES, LANES)]
                rv = plsc.load_gather(rec_v, [dv >> 7, dv & 127])
                plsc.addupdate_scatter(part_v, [sv >> 7, sv & 127], rv)
            return 0
        lax.fori_loop(0, EPT2 // LANES // 5, _c_step, 0)

        pltpu.sync_copy(part_v, sh_c.at[rows_v], add=True)
        plsc.subcore_barrier()

        NCHUNK = NPR // 8

        @pl.when(sid < NCHUNK)
        def _():
            pltpu.sync_copy(sh_c.at[pl.ds(sid * 8, 8)],
                            part_v.at[pl.ds(0, 8)])
            pltpu.sync_copy(part_v.at[pl.ds(0, 8)],
                            c_hbm.at[cid, pl.ds(sid * 8, 8)])

            @pl.when(cid == 0)
            def _():
                pltpu.sync_copy(rec_v.at[pl.ds(sid * 8, 8)],
                                r_hbm.at[pl.ds(sid * 8, 8)])

    return deg_c_kernel


def _make_spmm_kernel(ROWS_PER_W, NPR, H, TAIL0):
    NW = N_CORES * N_SUBCORES
    NP = NPR * 128
    RPT = NP // 128 // N_SUBCORES

    @functools.partial(
        pl.kernel,
        out_type=jax.ShapeDtypeStruct((N_CORES, NP, H), jnp.float32),
        mesh=_sc_mesh(),
        compiler_params=_SC_PARAMS,
        scratch_types=[
            pltpu.VMEM((2, 16, 128), jnp.int32),
            pltpu.VMEM((2, 16, 128), jnp.int32),
            pltpu.VMEM((128, H), jnp.float32),
            pltpu.VMEM((128, H), jnp.float32),
            pltpu.SemaphoreType.DMA,
            pltpu.SemaphoreType.DMA,
            pltpu.SemaphoreType.DMA,
            pltpu.SemaphoreType.DMA,
            pltpu.SemaphoreType.DMA,
            pltpu.VMEM_SHARED((NP, H), jnp.float32),
        ],
    )
    def spmm_kernel(src2d_hbm, dst2d_hbm, stail_hbm, dtail_hbm, x_hbm,
                    m_hbm,
                    src_v, dst_v, rows_a, rows_b,
                    sem_a, sem_b, sem_i, sem_sa, sem_sb, sh_m):
        cid = lax.axis_index("c")
        sid = lax.axis_index("s")
        w = sid * N_CORES + cid
        zeros = jnp.zeros((LANES,), jnp.float32)

        def _zero_buf(k, _):
            rows_a[k >> 3, pl.ds((k & 7) * LANES, LANES)] = zeros
            return 0
        lax.fori_loop(0, 128 * 8, _zero_buf, 0)
        for k in range(RPT):
            pltpu.sync_copy(rows_a, sh_m.at[pl.ds((sid * RPT + k) * 128, 128)])

        plsc.subcore_barrier()

        r0 = w * ROWS_PER_W
        NCH = ROWS_PER_W // 16
        nhalf = 8

        def _stage(ch, buf):
            cbase = r0 + ch * 16

            @pl.when(cbase < TAIL0)
            def _():
                pltpu.async_copy(src2d_hbm.at[pl.ds(cbase, 16)],
                                 src_v.at[buf], sem_i)
                pltpu.async_copy(dst2d_hbm.at[pl.ds(cbase, 16)],
                                 dst_v.at[buf], sem_i)

            @pl.when(cbase >= TAIL0)
            def _():
                pltpu.async_copy(stail_hbm.at[pl.ds(cbase - TAIL0, 16)],
                                 src_v.at[buf], sem_i)
                pltpu.async_copy(dtail_hbm.at[pl.ds(cbase - TAIL0, 16)],
                                 dst_v.at[buf], sem_i)

        def _drain_stage(buf):
            pltpu.make_async_copy(src2d_hbm.at[pl.ds(0, 16)],
                                  src_v.at[buf], sem_i).wait()
            pltpu.make_async_copy(dst2d_hbm.at[pl.ds(0, 16)],
                                  dst_v.at[buf], sem_i).wait()

        _stage(0, 0)
        for ch in range(NCH):
            b = ch % 2
            sv = src_v.at[b]
            dv = dst_v.at[b]
            _drain_stage(b)
            if ch + 1 < NCH:
                _stage(ch + 1, 1 - b)
            pltpu.async_copy(x_hbm.at[sv.at[0]], rows_a, sem_a)

            def _step(j2, _, sv=sv, dv=dv):
                j = j2 * 2
                pltpu.make_async_copy(
                    x_hbm.at[sv.at[j]], rows_a, sem_a).wait()
                cp_b = pltpu.async_copy(
                    x_hbm.at[sv.at[j + 1]], rows_b, sem_b)
                pltpu.sync_copy(rows_a, sh_m.at[dv.at[j]], add=True)

                @pl.when(j2 + 1 < nhalf)
                def _():
                    pltpu.async_copy(
                        x_hbm.at[sv.at[j + 2]], rows_a, sem_a)
                cp_b.wait()
                pltpu.sync_copy(rows_b, sh_m.at[dv.at[j + 1]], add=True)
                return 0

            lax.fori_loop(0, nhalf, _step, 0)

        plsc.subcore_barrier()

        for k in range(RPT):
            rbase = (sid * RPT + k) * 128
            pltpu.sync_copy(sh_m.at[pl.ds(rbase, 128)], rows_a)
            pltpu.sync_copy(rows_a, m_hbm.at[cid, pl.ds(rbase, 128)])

    return spmm_kernel


def _make_epilogue(nblocks, blk, N, H, OUT):
    SUB = blk // 8

    def body(x_ref, m0_ref, m1_ref, r_ref, c0_ref, c1_ref, b1_ref,
             w1s_ref, w1n_ref, w2s_ref, w2n_ref, b2_ref, wh_ref, bh_ref,
             out_ref, u_acc, v_acc):
        i = pl.program_id(0)

        @pl.when(i == 0)
        def _():
            u_acc[...] = jnp.zeros_like(u_acc)
            v_acc[...] = jnp.zeros_like(v_acc)

        mean = (m0_ref[0] + m1_ref[0]) * r_ref[...]
        h1 = jnp.maximum(
            jnp.dot(x_ref[...], w1s_ref[...],
                    preferred_element_type=jnp.float32)
            + jnp.dot(mean, w1n_ref[...],
                      preferred_element_type=jnp.float32)
            + b1_ref[...], 0.0)
        ccol = c0_ref[0] + c1_ref[0]
        u_acc[0:1, :] += jnp.sum(h1, axis=0, keepdims=True)
        v_acc[0:1, :] += jnp.sum(h1 * ccol, axis=0, keepdims=True)

        @pl.when(i == nblocks - 1)
        def _():
            u = u_acc[0:1, :]
            v = v_acc[0:1, :]
            pooled = (jnp.dot(u, w2s_ref[...], preferred_element_type=jnp.float32)
                      + jnp.dot(v, w2n_ref[...], preferred_element_type=jnp.float32)
                      ) * (1.0 / N) + b2_ref[...]
            out_ref[...] = (jnp.dot(pooled, wh_ref[...],
                                    preferred_element_type=jnp.float32)
                            + bh_ref[...])

    return body


def kernel(x, edge_index, batch, W1_self, W1_neigh, b1,
           W2_self, W2_neigh, b2, W_head, b_head):
    N, D = x.shape
    E = edge_index.shape[1]
    H = W1_self.shape[0]
    OUT = W_head.shape[0]

    NPR = (N + 127) // 128 + 1
    NP = NPR * 128

    NW = N_CORES * N_SUBCORES
    nrows = (E + 127) // 128
    rows_per_w = (nrows + NW - 1) // NW
    rows_per_w = (rows_per_w + 15) // 16 * 16
    E_pad = rows_per_w * NW * 128
    pad = E_pad - E
    TAIL0 = (nrows // 16) * 16
    TR = rows_per_w * NW - TAIL0
    ntail_real = E - TAIL0 * 128
    npad = TR * 128 - ntail_real
    pad_ids = lax.iota(jnp.int32, npad)
    junk0 = NP - 128
    src2d = edge_index[0].reshape(nrows, 128)
    dst2d = edge_index[1].reshape(nrows, 128)
    stail = jnp.concatenate(
        [edge_index[0, TAIL0 * 128:], pad_ids % N]).reshape(TR, 128)
    dtail = jnp.concatenate(
        [edge_index[1, TAIL0 * 128:],
         junk0 + (pad_ids % 128)]).reshape(TR, 128)

    r2d, c2d = _make_deg_c_kernel(E, NPR)(edge_index.reshape(2 * E))

    m_parts = _make_spmm_kernel(rows_per_w, NPR, H, TAIL0)(
        src2d, dst2d, stail, dtail, x)

    BLK = 2000
    nblocks = N // BLK
    r_col = r2d.reshape(NP)[:N].reshape(N, 1)
    c_cols = c2d.reshape(N_CORES, NP)[:, :N].reshape(N_CORES, N, 1)

    out = pl.pallas_call(
        _make_epilogue(nblocks, BLK, N, H, OUT),
        grid=(nblocks,),
        in_specs=[
            pl.BlockSpec((BLK, D), lambda i: (i, 0)),
            pl.BlockSpec((1, BLK, H), lambda i: (0, i, 0)),
            pl.BlockSpec((1, BLK, H), lambda i: (1, i, 0)),
            pl.BlockSpec((BLK, 1), lambda i: (i, 0)),
            pl.BlockSpec((1, BLK, 1), lambda i: (0, i, 0)),
            pl.BlockSpec((1, BLK, 1), lambda i: (1, i, 0)),
            pl.BlockSpec((1, H), lambda i: (0, 0)),
            pl.BlockSpec((D, H), lambda i: (0, 0)),
            pl.BlockSpec((H, H), lambda i: (0, 0)),
            pl.BlockSpec((H, H), lambda i: (0, 0)),
            pl.BlockSpec((H, H), lambda i: (0, 0)),
            pl.BlockSpec((1, H), lambda i: (0, 0)),
            pl.BlockSpec((H, OUT), lambda i: (0, 0)),
            pl.BlockSpec((1, OUT), lambda i: (0, 0)),
        ],
        out_specs=pl.BlockSpec((1, OUT), lambda i: (0, 0)),
        out_shape=jax.ShapeDtypeStruct((1, OUT), jnp.float32),
        scratch_shapes=[
            pltpu.VMEM((8, H), jnp.float32),
            pltpu.VMEM((8, H), jnp.float32),
        ],
    )(x, m_parts, m_parts, r_col, c_cols, c_cols,
      b1.reshape(1, H), W1_self.T, W1_neigh.T, W2_self.T, W2_neigh.T,
      b2.reshape(1, H), W_head.T, b_head.reshape(1, OUT))

    return out
